# baseline (speedup 1.0000x reference)
import jax
import jax.numpy as jnp
from jax import lax
from jax.experimental import pallas as pl
from jax.experimental.pallas import tpu as pltpu
from jax.experimental.pallas import tpu_sc as plsc

B, C, NUM, NCAT, V, D = 4096, 4, 13, 26, 100000, 32
NC, NS = 2, 16
NW = NC * NS
GU = 8


HA = 49920
HB = V - HA


def _sc_gather_body(s_catT_hbm, tablesT_hbm, out_hbm, bufA, bufB, idx0,
                    idx1, acc_v, sem_a, sem_b, sem_i, sem_o):
    cid = lax.axis_index("c")
    sid = lax.axis_index("s")
    wid = sid * NC + cid

    pltpu.async_copy(tablesT_hbm.at[0, wid].at[pl.ds(0, HA)], bufA, sem_a)

    def per_field(f, _):
        pltpu.make_async_copy(tablesT_hbm.at[f, wid].at[pl.ds(0, HA)],
                              bufA, sem_a).wait()
        pltpu.async_copy(tablesT_hbm.at[f, wid].at[pl.ds(HA, HB)],
                         bufB, sem_b)

        pltpu.async_copy(s_catT_hbm.at[f, 0], idx0, sem_i)
        for c in range(C):
            ib = idx0 if c % 2 == 0 else idx1
            nb = idx1 if c % 2 == 0 else idx0
            pltpu.make_async_copy(s_catT_hbm.at[f, c], ib, sem_i).wait()
            if c + 1 < C:
                pltpu.async_copy(s_catT_hbm.at[f, c + 1], nb, sem_i)

            def groupA(g, __, c=c, ib=ib):
                for u in range(GU):
                    sl = pl.ds((g * GU + u) * 16, 16)
                    idx = jnp.minimum(ib[sl], HA - 1)
                    acc_v[pl.ds(c * B + (g * GU + u) * 16, 16)] = (
                        plsc.load_gather(bufA, [idx]))
                return __

            lax.fori_loop(0, B // (16 * GU), groupA, None)

        pltpu.make_async_copy(tablesT_hbm.at[f, wid].at[pl.ds(HA, HB)],
                              bufB, sem_b).wait()

        @pl.when(f + 1 < NCAT)
        def _start_next():
            pltpu.async_copy(tablesT_hbm.at[f + 1, wid].at[pl.ds(0, HA)],
                             bufA, sem_a)

        pltpu.async_copy(s_catT_hbm.at[f, 0], idx0, sem_i)
        for c in range(C):
            ib = idx0 if c % 2 == 0 else idx1
            nb = idx1 if c % 2 == 0 else idx0
            pltpu.make_async_copy(s_catT_hbm.at[f, c], ib, sem_i).wait()
            if c + 1 < C:
                pltpu.async_copy(s_catT_hbm.at[f, c + 1], nb, sem_i)

            def groupB(g, __, c=c, ib=ib):
                for u in range(GU):
                    sl = pl.ds((g * GU + u) * 16, 16)
                    asl = pl.ds(c * B + (g * GU + u) * 16, 16)
                    raw = ib[sl]
                    idx = jnp.maximum(raw - HA, 0)
                    hi = plsc.load_gather(bufB, [idx])
                    acc_v[asl] = jnp.where(raw < HA, acc_v[asl], hi)
                return __

            lax.fori_loop(0, B // (16 * GU), groupB, None)
            pltpu.async_copy(acc_v.at[pl.ds(c * B, B)], out_hbm.at[c, f, wid],
                             sem_o)

        for c in range(C):
            pltpu.make_async_copy(acc_v.at[pl.ds(c * B, B)],
                                  out_hbm.at[c, f, wid], sem_o).wait()
        return _

    lax.fori_loop(0, NCAT, per_field, None)


@jax.jit
def _sc_gather(s_catT, tablesT):
    mesh = plsc.VectorSubcoreMesh(core_axis_name="c", subcore_axis_name="s")
    return pl.kernel(
        _sc_gather_body,
        out_type=jax.ShapeDtypeStruct((C, NCAT, D, B), jnp.float32),
        mesh=mesh,
        scratch_types=[
            pltpu.VMEM((HA,), jnp.float32),
            pltpu.VMEM((HB,), jnp.float32),
            pltpu.VMEM((B,), jnp.int32),
            pltpu.VMEM((B,), jnp.int32),
            pltpu.VMEM((C * B,), jnp.float32),
            pltpu.SemaphoreType.DMA,
            pltpu.SemaphoreType.DMA,
            pltpu.SemaphoreType.DMA,
            pltpu.SemaphoreType.DMA,
        ],
        compiler_params=pltpu.CompilerParams(needs_layout_passes=False),
    )(s_catT, tablesT)


BBLK = 512
KD = (NUM + NCAT) * D


def _assemble_body(s_contT_ref, w_ref, nan_ref, cat_ref, out_ref):
    w = w_ref[...]
    nan_e = nan_ref[...]
    for c in range(C):
        sc = s_contT_ref[:, c, :]
        ne = sc[:, None, :] * w[:, :, None]
        nan_mask = jnp.isnan(sc)[:, None, :]
        enc = jnp.where(nan_mask, nan_e[:, :, None], ne)
        out_ref[pl.ds(c * KD, NUM * D), :] = enc.reshape(NUM * D, BBLK)
        cat = cat_ref[c]
        out_ref[pl.ds(c * KD + NUM * D, NCAT * D), :] = cat.reshape(
            NCAT * D, BBLK)


@jax.jit
def _assemble(s_contT, num_W, nan_embs, m_cat):
    return pl.pallas_call(
        _assemble_body,
        grid=(B // BBLK,),
        in_specs=[
            pl.BlockSpec((NUM, C, BBLK), lambda i: (0, 0, i)),
            pl.BlockSpec((NUM, D), lambda i: (0, 0)),
            pl.BlockSpec((NUM, D), lambda i: (0, 0)),
            pl.BlockSpec((C, NCAT, D, BBLK), lambda i: (0, 0, 0, i)),
        ],
        out_specs=pl.BlockSpec((C * KD, BBLK), lambda i: (0, i)),
        out_shape=jax.ShapeDtypeStruct((C * KD, B), jnp.float32),
    )(s_contT, num_W, nan_embs, m_cat)


def kernel(s_cont, s_cat, cat_tables, num_W, nan_embs):
    tablesT = jnp.transpose(cat_tables, (0, 2, 1))
    s_catT = jnp.transpose(s_cat, (2, 1, 0))
    s_contT = jnp.transpose(s_cont, (2, 1, 0))
    m_cat = _sc_gather(s_catT, tablesT)
    m2 = _assemble(s_contT, num_W, nan_embs, m_cat)
    out = jnp.transpose(m2.reshape(C, NUM + NCAT, D, B), (3, 0, 1, 2))
    return out


# speedup vs baseline: 1.2766x; 1.2766x over previous
import jax
import jax.numpy as jnp
from jax import lax
from jax.experimental import pallas as pl
from jax.experimental.pallas import tpu as pltpu
from jax.experimental.pallas import tpu_sc as plsc

B, C, NUM, NCAT, V, D = 4096, 4, 13, 26, 100000, 32
NC, NS = 2, 16
NW = NC * NS
GU = 8


def _sc_gather_body(s_catT_hbm, tablesT_hbm, out_hbm, slice_v, idx0, idx1,
                    val0, val1, sem_s, sem_i, sem_o):
    cid = lax.axis_index("c")
    sid = lax.axis_index("s")
    wid = sid * NC + cid

    pltpu.sync_copy(tablesT_hbm.at[0, wid], slice_v)

    def per_field(f, _):
        pltpu.async_copy(s_catT_hbm.at[f, 0], idx0, sem_i)
        for c in range(C):
            ib = idx0 if c % 2 == 0 else idx1
            nb = idx1 if c % 2 == 0 else idx0
            vb = val0 if c % 2 == 0 else val1
            pltpu.make_async_copy(s_catT_hbm.at[f, c], ib, sem_i).wait()
            if c + 1 < C:
                pltpu.async_copy(s_catT_hbm.at[f, c + 1], nb, sem_i)
            if c >= 2:
                pltpu.make_async_copy(vb, out_hbm.at[c - 2, f, wid],
                                      sem_o).wait()

            def per_group(g, __, ib=ib, vb=vb):
                for u in range(GU):
                    sl = pl.ds((g * GU + u) * 16, 16)
                    vb[sl] = plsc.load_gather(slice_v, [ib[sl]])
                return __

            lax.fori_loop(0, B // (16 * GU), per_group, None)
            pltpu.async_copy(vb, out_hbm.at[c, f, wid], sem_o)

        pltpu.make_async_copy(val0, out_hbm.at[2, f, wid], sem_o).wait()
        pltpu.make_async_copy(val1, out_hbm.at[3, f, wid], sem_o).wait()

        @pl.when(f + 1 < NCAT)
        def _next_slice():
            pltpu.sync_copy(tablesT_hbm.at[f + 1, wid], slice_v)

        return _

    lax.fori_loop(0, NCAT, per_field, None)


@jax.jit
def _sc_gather(s_catT, tablesT):
    mesh = plsc.VectorSubcoreMesh(core_axis_name="c", subcore_axis_name="s")
    return pl.kernel(
        _sc_gather_body,
        out_type=jax.ShapeDtypeStruct((C, NCAT, D, B), jnp.float32),
        mesh=mesh,
        scratch_types=[
            pltpu.VMEM((V,), jnp.float32),
            pltpu.VMEM((B,), jnp.int32),
            pltpu.VMEM((B,), jnp.int32),
            pltpu.VMEM((B,), jnp.float32),
            pltpu.VMEM((B,), jnp.float32),
            pltpu.SemaphoreType.DMA,
            pltpu.SemaphoreType.DMA,
            pltpu.SemaphoreType.DMA,
        ],
        compiler_params=pltpu.CompilerParams(needs_layout_passes=False),
    )(s_catT, tablesT)


BBLK = 512
KD = (NUM + NCAT) * D


def _assemble_body(s_contT_ref, w_ref, nan_ref, cat_ref, out_ref):
    w = w_ref[...]
    nan_e = nan_ref[...]
    for c in range(C):
        sc = s_contT_ref[:, c, :]
        ne = sc[:, None, :] * w[:, :, None]
        nan_mask = jnp.isnan(sc)[:, None, :]
        enc = jnp.where(nan_mask, nan_e[:, :, None], ne)
        out_ref[pl.ds(c * KD, NUM * D), :] = enc.reshape(NUM * D, BBLK)
        cat = cat_ref[c]
        out_ref[pl.ds(c * KD + NUM * D, NCAT * D), :] = cat.reshape(
            NCAT * D, BBLK)


@jax.jit
def _assemble(s_contT, num_W, nan_embs, m_cat):
    return pl.pallas_call(
        _assemble_body,
        grid=(B // BBLK,),
        in_specs=[
            pl.BlockSpec((NUM, C, BBLK), lambda i: (0, 0, i)),
            pl.BlockSpec((NUM, D), lambda i: (0, 0)),
            pl.BlockSpec((NUM, D), lambda i: (0, 0)),
            pl.BlockSpec((C, NCAT, D, BBLK), lambda i: (0, 0, 0, i)),
        ],
        out_specs=pl.BlockSpec((C * KD, BBLK), lambda i: (0, i)),
        out_shape=jax.ShapeDtypeStruct((C * KD, B), jnp.float32),
    )(s_contT, num_W, nan_embs, m_cat)


def kernel(s_cont, s_cat, cat_tables, num_W, nan_embs):
    tablesT = jnp.transpose(cat_tables, (0, 2, 1))
    s_catT = jnp.transpose(s_cat, (2, 1, 0))
    s_contT = jnp.transpose(s_cont, (2, 1, 0))
    m_cat = _sc_gather(s_catT, tablesT)
    m2 = _assemble(s_contT, num_W, nan_embs, m_cat)
    out = jnp.transpose(m2.reshape(C, NUM + NCAT, D, B), (3, 0, 1, 2))
    return out


# speedup vs baseline: 1.3598x; 1.0652x over previous
import jax
import jax.numpy as jnp
from jax import lax
from jax.experimental import pallas as pl
from jax.experimental.pallas import tpu as pltpu
from jax.experimental.pallas import tpu_sc as plsc

B, C, NUM, NCAT, V, D = 4096, 4, 13, 26, 100000, 32
NC, NS = 2, 16
NW = NC * NS
GU = 8
HA = 49920
HB = V - HA


def _start_slice(tablesT_hbm, slice_v, f, wid, sem_s, sem_s2):
    del sem_s2
    pltpu.async_copy(tablesT_hbm.at[f, wid], slice_v, sem_s)


def _wait_slice(tablesT_hbm, slice_v, f, wid, sem_s, sem_s2):
    del sem_s2
    pltpu.make_async_copy(tablesT_hbm.at[f, wid], slice_v, sem_s).wait()


def _sc_gather_body(s_catT_hbm, tablesT_hbm, out_hbm, slice_v, idx0, idx1,
                    val0, val1, sem_s, sem_s2, sem_i, sem_o):
    cid = lax.axis_index("c")
    sid = lax.axis_index("s")
    wid = sid * NC + cid

    _start_slice(tablesT_hbm, slice_v, 0, wid, sem_s, sem_s2)

    def per_field(f, _):
        pltpu.async_copy(s_catT_hbm.at[f, 0], idx0, sem_i)
        _wait_slice(tablesT_hbm, slice_v, f, wid, sem_s, sem_s2)
        for c in range(C):
            ib = idx0 if c % 2 == 0 else idx1
            nb = idx1 if c % 2 == 0 else idx0
            vb = val0 if c % 2 == 0 else val1
            pltpu.make_async_copy(s_catT_hbm.at[f, c], ib, sem_i).wait()
            if c + 1 < C:
                pltpu.async_copy(s_catT_hbm.at[f, c + 1], nb, sem_i)
            if c >= 2:
                pltpu.make_async_copy(vb, out_hbm.at[c - 2, f, wid],
                                      sem_o).wait()

            def per_group(g, __, ib=ib, vb=vb):
                for u in range(GU):
                    sl = pl.ds((g * GU + u) * 16, 16)
                    vb[sl] = plsc.load_gather(slice_v, [ib[sl]])
                return __

            lax.fori_loop(0, B // (16 * GU), per_group, None)
            pltpu.async_copy(vb, out_hbm.at[c, f, wid], sem_o)

        @pl.when(f + 1 < NCAT)
        def _next_slice():
            _start_slice(tablesT_hbm, slice_v, f + 1, wid, sem_s, sem_s2)

        pltpu.make_async_copy(val0, out_hbm.at[2, f, wid], sem_o).wait()
        pltpu.make_async_copy(val1, out_hbm.at[3, f, wid], sem_o).wait()
        return _

    lax.fori_loop(0, NCAT, per_field, None)


@jax.jit
def _sc_gather(s_catT, tablesT):
    mesh = plsc.VectorSubcoreMesh(core_axis_name="c", subcore_axis_name="s")
    return pl.kernel(
        _sc_gather_body,
        out_type=jax.ShapeDtypeStruct((C, NCAT, D, B), jnp.float32),
        mesh=mesh,
        scratch_types=[
            pltpu.VMEM((V,), jnp.float32),
            pltpu.VMEM((B,), jnp.int32),
            pltpu.VMEM((B,), jnp.int32),
            pltpu.VMEM((B,), jnp.float32),
            pltpu.VMEM((B,), jnp.float32),
            pltpu.SemaphoreType.DMA,
            pltpu.SemaphoreType.DMA,
            pltpu.SemaphoreType.DMA,
            pltpu.SemaphoreType.DMA,
        ],
        compiler_params=pltpu.CompilerParams(needs_layout_passes=False),
    )(s_catT, tablesT)


BBLK = 512
KD = (NUM + NCAT) * D


def _assemble_body(s_contT_ref, w_ref, nan_ref, cat_ref, out_ref):
    w = w_ref[...]
    nan_e = nan_ref[...]
    for c in range(C):
        sc = s_contT_ref[:, c, :]
        ne = sc[:, None, :] * w[:, :, None]
        nan_mask = jnp.isnan(sc)[:, None, :]
        enc = jnp.where(nan_mask, nan_e[:, :, None], ne)
        out_ref[pl.ds(c * KD, NUM * D), :] = enc.reshape(NUM * D, BBLK)
        cat = cat_ref[c]
        out_ref[pl.ds(c * KD + NUM * D, NCAT * D), :] = cat.reshape(
            NCAT * D, BBLK)


@jax.jit
def _assemble(s_contT, num_W, nan_embs, m_cat):
    return pl.pallas_call(
        _assemble_body,
        grid=(B // BBLK,),
        in_specs=[
            pl.BlockSpec((NUM, C, BBLK), lambda i: (0, 0, i)),
            pl.BlockSpec((NUM, D), lambda i: (0, 0)),
            pl.BlockSpec((NUM, D), lambda i: (0, 0)),
            pl.BlockSpec((C, NCAT, D, BBLK), lambda i: (0, 0, 0, i)),
        ],
        out_specs=pl.BlockSpec((C * KD, BBLK), lambda i: (0, i)),
        out_shape=jax.ShapeDtypeStruct((C * KD, B), jnp.float32),
    )(s_contT, num_W, nan_embs, m_cat)


def kernel(s_cont, s_cat, cat_tables, num_W, nan_embs):
    tablesT = jnp.transpose(cat_tables, (0, 2, 1))
    s_catT = jnp.transpose(s_cat, (2, 1, 0))
    s_contT = jnp.transpose(s_cont, (2, 1, 0))
    m_cat = _sc_gather(s_catT, tablesT)
    m2 = _assemble(s_contT, num_W, nan_embs, m_cat)
    out = jnp.transpose(m2.reshape(C, NUM + NCAT, D, B), (3, 0, 1, 2))
    return out


# speedup vs baseline: 1.3627x; 1.0021x over previous
import jax
import jax.numpy as jnp
from jax import lax
from jax.experimental import pallas as pl
from jax.experimental.pallas import tpu as pltpu
from jax.experimental.pallas import tpu_sc as plsc

B, C, NUM, NCAT, V, D = 4096, 4, 13, 26, 100000, 32
NC, NS = 2, 16
NW = NC * NS
GU = 16
HA = 49920
HB = V - HA


def _start_slice(tablesT_hbm, slice_v, f, wid, sem_s, sem_s2):
    del sem_s2
    pltpu.async_copy(tablesT_hbm.at[f, wid], slice_v, sem_s)


def _wait_slice(tablesT_hbm, slice_v, f, wid, sem_s, sem_s2):
    del sem_s2
    pltpu.make_async_copy(tablesT_hbm.at[f, wid], slice_v, sem_s).wait()


def _sc_gather_body(s_catT_hbm, tablesT_hbm, out_hbm, slice_v, idx0, idx1,
                    val0, val1, sem_s, sem_s2, sem_i, sem_o):
    cid = lax.axis_index("c")
    sid = lax.axis_index("s")
    wid = sid * NC + cid

    _start_slice(tablesT_hbm, slice_v, 0, wid, sem_s, sem_s2)

    def per_field(f, _):
        pltpu.async_copy(s_catT_hbm.at[f, 0], idx0, sem_i)
        _wait_slice(tablesT_hbm, slice_v, f, wid, sem_s, sem_s2)
        for c in range(C):
            ib = idx0 if c % 2 == 0 else idx1
            nb = idx1 if c % 2 == 0 else idx0
            vb = val0 if c % 2 == 0 else val1
            pltpu.make_async_copy(s_catT_hbm.at[f, c], ib, sem_i).wait()
            if c + 1 < C:
                pltpu.async_copy(s_catT_hbm.at[f, c + 1], nb, sem_i)
            if c >= 2:
                pltpu.make_async_copy(vb, out_hbm.at[c - 2, f, wid],
                                      sem_o).wait()

            def per_group(g, __, ib=ib, vb=vb):
                for u in range(GU):
                    sl = pl.ds((g * GU + u) * 16, 16)
                    vb[sl] = plsc.load_gather(slice_v, [ib[sl]])
                return __

            lax.fori_loop(0, B // (16 * GU), per_group, None)
            pltpu.async_copy(vb, out_hbm.at[c, f, wid], sem_o)

        @pl.when(f + 1 < NCAT)
        def _next_slice():
            _start_slice(tablesT_hbm, slice_v, f + 1, wid, sem_s, sem_s2)

        pltpu.make_async_copy(val0, out_hbm.at[2, f, wid], sem_o).wait()
        pltpu.make_async_copy(val1, out_hbm.at[3, f, wid], sem_o).wait()
        return _

    lax.fori_loop(0, NCAT, per_field, None)


@jax.jit
def _sc_gather(s_catT, tablesT):
    mesh = plsc.VectorSubcoreMesh(core_axis_name="c", subcore_axis_name="s")
    return pl.kernel(
        _sc_gather_body,
        out_type=jax.ShapeDtypeStruct((C, NCAT, D, B), jnp.float32),
        mesh=mesh,
        scratch_types=[
            pltpu.VMEM((V,), jnp.float32),
            pltpu.VMEM((B,), jnp.int32),
            pltpu.VMEM((B,), jnp.int32),
            pltpu.VMEM((B,), jnp.float32),
            pltpu.VMEM((B,), jnp.float32),
            pltpu.SemaphoreType.DMA,
            pltpu.SemaphoreType.DMA,
            pltpu.SemaphoreType.DMA,
            pltpu.SemaphoreType.DMA,
        ],
        compiler_params=pltpu.CompilerParams(needs_layout_passes=False),
    )(s_catT, tablesT)


BBLK = 512
KD = (NUM + NCAT) * D


def _assemble_body(s_contT_ref, w_ref, nan_ref, cat_ref, out_ref):
    w = w_ref[...]
    nan_e = nan_ref[...]
    for c in range(C):
        sc = s_contT_ref[:, c, :]
        ne = sc[:, None, :] * w[:, :, None]
        nan_mask = jnp.isnan(sc)[:, None, :]
        enc = jnp.where(nan_mask, nan_e[:, :, None], ne)
        out_ref[pl.ds(c * KD, NUM * D), :] = enc.reshape(NUM * D, BBLK)
        cat = cat_ref[c]
        out_ref[pl.ds(c * KD + NUM * D, NCAT * D), :] = cat.reshape(
            NCAT * D, BBLK)


@jax.jit
def _assemble(s_contT, num_W, nan_embs, m_cat):
    return pl.pallas_call(
        _assemble_body,
        grid=(B // BBLK,),
        in_specs=[
            pl.BlockSpec((NUM, C, BBLK), lambda i: (0, 0, i)),
            pl.BlockSpec((NUM, D), lambda i: (0, 0)),
            pl.BlockSpec((NUM, D), lambda i: (0, 0)),
            pl.BlockSpec((C, NCAT, D, BBLK), lambda i: (0, 0, 0, i)),
        ],
        out_specs=pl.BlockSpec((C * KD, BBLK), lambda i: (0, i)),
        out_shape=jax.ShapeDtypeStruct((C * KD, B), jnp.float32),
    )(s_contT, num_W, nan_embs, m_cat)


def kernel(s_cont, s_cat, cat_tables, num_W, nan_embs):
    tablesT = jnp.transpose(cat_tables, (0, 2, 1))
    s_catT = jnp.transpose(s_cat, (2, 1, 0))
    s_contT = jnp.transpose(s_cont, (2, 1, 0))
    m_cat = _sc_gather(s_catT, tablesT)
    m2 = _assemble(s_contT, num_W, nan_embs, m_cat)
    out = jnp.transpose(m2.reshape(C, NUM + NCAT, D, B), (3, 0, 1, 2))
    return out
